# Initial kernel scaffold; baseline (speedup 1.0000x reference)
import jax
import jax.numpy as jnp
from jax.experimental import pallas as pl


def kernel(x, Wr, Wqkv, Wproj, bproj, W1, b1, W2, b2, ln1_w, ln1_b, ln2_w, ln2_b, alpha):
    raise NotImplementedError("write your pallas kernel here")


# speedup vs baseline: 1.4237x; 1.4237x over previous
import jax
import jax.numpy as jnp
from jax.experimental import pallas as pl

D = 768
E = 8
NHEADS = 12
HD = 64
HID = 3072
N = 2048
CAPS = [int(c * N) for c in [0.3, 0.2, 0.15, 0.1, 0.1, 0.05, 0.05, 0.05]]
QT = 512
TT = 256


def _router_kernel(xT_ref, WrT_ref, probs_ref, emask_ref, eid_ref, ep_ref):
    logitsT = jnp.dot(WrT_ref[...], xT_ref[...],
                      preferred_element_type=jnp.float32)
    mx = jnp.max(logitsT, axis=0, keepdims=True)
    ex = jnp.exp(logitsT - mx)
    pT = ex / jnp.sum(ex, axis=0, keepdims=True)

    assigned = jnp.zeros((1, N), dtype=jnp.bool_)
    eidT = jnp.zeros((1, N), dtype=jnp.int32)
    for e in range(E - 1, 0, -1):
        k = CAPS[e]
        iv = jax.lax.bitcast_convert_type(pT[e:e + 1, :], jnp.int32)
        iv = jnp.where(assigned, -1, iv)

        def body(_, lohi, iv=iv, k=k):
            lo, hi = lohi
            mid = lo + (hi - lo + 1) // 2
            cnt = jnp.sum((iv >= mid).astype(jnp.int32))
            big = cnt >= k
            return (jnp.where(big, mid, lo), jnp.where(big, hi, mid - 1))

        lo, _ = jax.lax.fori_loop(
            0, 31, body, (jnp.int32(0), jnp.int32(0x40000000)))
        sel = iv >= lo
        eidT = jnp.where(sel, e, eidT)
        assigned = jnp.logical_or(assigned, sel)

    srow = jax.lax.broadcasted_iota(jnp.int32, (E, N), 0)
    maskT = (srow == eidT).astype(jnp.float32)

    probs = jnp.transpose(pT)
    emask = jnp.transpose(maskT)
    probs_ref[...] = probs
    emask_ref[...] = emask
    lane = jax.lax.broadcasted_iota(jnp.int32, (N, E), 1).astype(jnp.float32)
    eid_ref[...] = jnp.sum(emask * lane, axis=1, keepdims=True).astype(jnp.int32)
    ep_ref[...] = jnp.sum(emask * probs, axis=1, keepdims=True)


def _qkv_kernel(x_ref, Wqkv_ref, w_ref, b_ref, eid_ref, qkv_ref):
    x = x_ref[...]
    mu = jnp.mean(x, axis=1, keepdims=True)
    xc = x - mu
    var = jnp.mean(xc * xc, axis=1, keepdims=True)
    h = xc * jax.lax.rsqrt(var + 1e-5) * w_ref[...] + b_ref[...]
    qkv = jnp.dot(h, Wqkv_ref[...], preferred_element_type=jnp.float32)
    bound = 96 * (eid_ref[...] + 1)
    col = jax.lax.broadcasted_iota(jnp.int32, (TT, 3 * D), 1)
    m3 = ((col < bound)
          | ((col >= D) & (col < bound + D))
          | ((col >= 2 * D) & (col < bound + 2 * D)))
    qkv_ref[...] = jnp.where(m3, qkv, 0.0)


def _attn_kernel(q_ref, k_ref, v_ref, o_ref):
    q = q_ref[0]
    k = k_ref[0]
    s = jax.lax.dot_general(q, k, (((1,), (1,)), ((), ())),
                            preferred_element_type=jnp.float32)
    s = s * (HD ** -0.5)
    s = s - jnp.max(s, axis=1, keepdims=True)
    p = jnp.exp(s)
    p = p / jnp.sum(p, axis=1, keepdims=True)
    o_ref[0] = jnp.dot(p, v_ref[0], preferred_element_type=jnp.float32)


def _post_kernel(o_ref, x_ref, eid_ref, ep_ref, Wproj_ref, bproj_ref,
                 w2_ref, b2w_ref, W1_ref, b1_ref, W2_ref, b2_ref, alpha_ref,
                 out_ref):
    eid = eid_ref[...]
    bound = 96 * (eid + 1)
    col = jax.lax.broadcasted_iota(jnp.int32, (TT, D), 1)
    m = col < bound
    colh = jax.lax.broadcasted_iota(jnp.int32, (TT, HID), 1)
    mh = colh < 384 * (eid + 1)

    o = jnp.where(m, o_ref[...], 0.0)
    o1 = jnp.dot(o, Wproj_ref[...], preferred_element_type=jnp.float32)
    o1 = o1 + bproj_ref[...]
    o1 = jnp.where(m, o1, 0.0)
    z = x_ref[...] + o1

    mu = jnp.mean(z, axis=1, keepdims=True)
    zc = z - mu
    var = jnp.mean(zc * zc, axis=1, keepdims=True)
    h2 = zc * jax.lax.rsqrt(var + 1e-5) * w2_ref[...] + b2w_ref[...]
    h2 = jnp.where(m, h2, 0.0)

    hid = jnp.dot(h2, W1_ref[...], preferred_element_type=jnp.float32)
    hid = hid + b1_ref[...]
    hid = hid * 0.5 * (1.0 + jax.lax.erf(hid * (2.0 ** -0.5)))
    hid = jnp.where(mh, hid, 0.0)

    zp = jnp.dot(hid, W2_ref[...], preferred_element_type=jnp.float32)
    zp = zp + b2_ref[...]
    zp = jnp.where(m, zp, 0.0)

    scale = alpha_ref[0, 0] * ep_ref[...] + 1.0
    out_ref[...] = z + scale * zp


def _full(shape):
    return pl.BlockSpec(shape, lambda *a: tuple(0 for _ in shape))


def kernel(x, Wr, Wqkv, Wproj, bproj, W1, b1, W2, b2,
           ln1_w, ln1_b, ln2_w, ln2_b, alpha):
    x2 = x[0]
    xT = jnp.transpose(x2)
    WrT = jnp.transpose(Wr)

    probs, emask, eid, ep = pl.pallas_call(
        _router_kernel,
        out_shape=[
            jax.ShapeDtypeStruct((N, E), jnp.float32),
            jax.ShapeDtypeStruct((N, E), jnp.float32),
            jax.ShapeDtypeStruct((N, 1), jnp.int32),
            jax.ShapeDtypeStruct((N, 1), jnp.float32),
        ],
    )(xT, WrT)

    qkv = pl.pallas_call(
        _qkv_kernel,
        grid=(N // TT,),
        in_specs=[
            pl.BlockSpec((TT, D), lambda i: (i, 0)),
            _full((D, 3 * D)),
            _full((1, D)),
            _full((1, D)),
            pl.BlockSpec((TT, 1), lambda i: (i, 0)),
        ],
        out_specs=pl.BlockSpec((TT, 3 * D), lambda i: (i, 0)),
        out_shape=jax.ShapeDtypeStruct((N, 3 * D), jnp.float32),
    )(x2, Wqkv, ln1_w.reshape(1, D), ln1_b.reshape(1, D), eid)

    qkv3 = jnp.transpose(qkv.reshape(N, 3 * NHEADS, HD), (1, 0, 2))

    o3 = pl.pallas_call(
        _attn_kernel,
        grid=(NHEADS, N // QT),
        in_specs=[
            pl.BlockSpec((1, QT, HD), lambda h, i: (h, i, 0)),
            pl.BlockSpec((1, N, HD), lambda h, i: (NHEADS + h, 0, 0)),
            pl.BlockSpec((1, N, HD), lambda h, i: (2 * NHEADS + h, 0, 0)),
        ],
        out_specs=pl.BlockSpec((1, QT, HD), lambda h, i: (h, i, 0)),
        out_shape=jax.ShapeDtypeStruct((NHEADS, N, HD), jnp.float32),
    )(qkv3, qkv3, qkv3)
    o = jnp.transpose(o3, (1, 0, 2)).reshape(N, D)

    out = pl.pallas_call(
        _post_kernel,
        grid=(N // TT,),
        in_specs=[
            pl.BlockSpec((TT, D), lambda i: (i, 0)),
            pl.BlockSpec((TT, D), lambda i: (i, 0)),
            pl.BlockSpec((TT, 1), lambda i: (i, 0)),
            pl.BlockSpec((TT, 1), lambda i: (i, 0)),
            _full((D, D)),
            _full((1, D)),
            _full((1, D)),
            _full((1, D)),
            _full((D, HID)),
            _full((1, HID)),
            _full((HID, D)),
            _full((1, D)),
            _full((1, 1)),
        ],
        out_specs=pl.BlockSpec((TT, D), lambda i: (i, 0)),
        out_shape=jax.ShapeDtypeStruct((N, D), jnp.float32),
    )(o, x2, eid, ep, Wproj, bproj.reshape(1, D),
      ln2_w.reshape(1, D), ln2_b.reshape(1, D), W1, b1.reshape(1, HID),
      W2, b2.reshape(1, D), alpha.reshape(1, 1))

    return out[None], emask[None], probs[None]
